# baseline (speedup 1.0000x reference)
import functools

import jax
import jax.numpy as jnp
from jax import lax
from jax.experimental import pallas as pl
from jax.experimental.pallas import tpu as pltpu
from jax.experimental.pallas import tpu_sc as plsc

_N0 = 16384
_N1 = 26
_N1P = 32
_DP = 128
_B = _N0 * _N1P
_NC = 2
_NS = 16
_NW = _NC * _NS
_B_PER_W = _B // _NW
_CHUNK = 128
_N_CHUNKS = _B_PER_W // _CHUNK
_NBUF = 4
_NGROUPS = _N_CHUNKS // _NBUF

_mesh = plsc.VectorSubcoreMesh(core_axis_name="c", subcore_axis_name="s")


@functools.partial(
    pl.kernel,
    mesh=_mesh,
    out_type=jax.ShapeDtypeStruct((_B, _DP), jnp.float32),
    scratch_types=[
        pltpu.VMEM((_B_PER_W,), jnp.int32),
        pltpu.VMEM((_NBUF, _CHUNK, _DP), jnp.float32),
    ]
    + [pltpu.SemaphoreType.DMA] * (2 * _NBUF),
    compiler_params=pltpu.CompilerParams(use_tc_tiling_on_sc=True),
)
def _gather_kernel(idx_hbm, table_hbm, out_hbm, idx_v, rows_v, *sems):
    gsem = sems[:_NBUF]
    osem = sems[_NBUF:]
    wid = lax.axis_index("s") * _NC + lax.axis_index("c")
    base = wid * _B_PER_W
    pltpu.sync_copy(idx_hbm.at[pl.ds(base, _B_PER_W)], idx_v)

    def g_copy(ci, b):
        return pltpu.make_async_copy(
            table_hbm.at[idx_v.at[pl.ds(ci * _CHUNK, _CHUNK)]],
            rows_v.at[b],
            gsem[b],
        )

    def o_copy(ci, b):
        return pltpu.make_async_copy(
            rows_v.at[b],
            out_hbm.at[pl.ds(base + ci * _CHUNK, _CHUNK)],
            osem[b],
        )

    for b in range(_NBUF):
        g_copy(b, b).start()

    def body(g, carry):
        ci0 = g * _NBUF
        for b in range(_NBUF):
            g_copy(ci0 + b, b).wait()
            o_copy(ci0 + b, b).start()
        for b in range(_NBUF):
            o_copy(ci0 + b, b).wait()
            g_copy(ci0 + _NBUF + b, b).start()
        return carry

    lax.fori_loop(0, _NGROUPS - 1, body, 0)

    ci0 = (_NGROUPS - 1) * _NBUF
    for b in range(_NBUF):
        g_copy(ci0 + b, b).wait()
        o_copy(ci0 + b, b).start()
    for b in range(_NBUF):
        o_copy(ci0 + b, b).wait()


def kernel(idx, action_embedding):
    table_pad = jnp.pad(action_embedding, ((0, 0), (0, _DP - 64)))
    idx_pad = jnp.pad(idx, ((0, 0), (0, _N1P - _N1))).reshape(-1)
    flat = _gather_kernel(idx_pad, table_pad)
    return flat.reshape(_N0, _N1P, _DP)[:, :_N1, :64]


# speedup vs baseline: 5.8394x; 5.8394x over previous
import functools

import jax
import jax.numpy as jnp
from jax import lax
from jax.experimental import pallas as pl
from jax.experimental.pallas import tpu as pltpu
from jax.experimental.pallas import tpu_sc as plsc

_N0 = 16384
_N1 = 26
_N1P = 32
_DP = 128
_B = _N0 * _N1P
_NC = 2
_NS = 16
_NW = _NC * _NS
_B_PER_W = _B // _NW
_CHUNK = 128
_N_CHUNKS = _B_PER_W // _CHUNK
_NBUF = 4
_NGROUPS = _N_CHUNKS // _NBUF

_mesh = plsc.VectorSubcoreMesh(core_axis_name="c", subcore_axis_name="s")


@functools.partial(
    pl.kernel,
    mesh=_mesh,
    out_type=jax.ShapeDtypeStruct((_B, _DP), jnp.float32),
    scratch_types=[
        pltpu.VMEM((_B_PER_W,), jnp.int32),
        pltpu.VMEM((_NBUF, _CHUNK, _DP), jnp.float32),
    ]
    + [pltpu.SemaphoreType.DMA] * (2 * _NBUF),
    compiler_params=pltpu.CompilerParams(use_tc_tiling_on_sc=True),
)
def _gather_kernel(idx_hbm, table_hbm, out_hbm, idx_v, rows_v, *sems):
    gsem = sems[:_NBUF]
    osem = sems[_NBUF:]
    wid = lax.axis_index("s") * _NC + lax.axis_index("c")
    base = wid * _B_PER_W
    pltpu.sync_copy(idx_hbm.at[pl.ds(base, _B_PER_W)], idx_v)

    def g_copy(ci, b):
        return pltpu.make_async_copy(
            table_hbm.at[idx_v.at[pl.ds(ci * _CHUNK, _CHUNK)]],
            rows_v.at[b],
            gsem[b],
        )

    def o_copy(ci, b):
        return pltpu.make_async_copy(
            rows_v.at[b],
            out_hbm.at[pl.ds(base + ci * _CHUNK, _CHUNK)],
            osem[b],
        )

    for b in range(_NBUF):
        g_copy(b, b).start()

    def body(g, carry):
        ci0 = g * _NBUF
        for b in range(_NBUF):
            g_copy(ci0 + b, b).wait()
            o_copy(ci0 + b, b).start()
        for b in range(_NBUF):
            o_copy(ci0 + b, b).wait()
            g_copy(ci0 + _NBUF + b, b).start()
        return carry

    lax.fori_loop(0, _NGROUPS - 1, body, 0)

    ci0 = (_NGROUPS - 1) * _NBUF
    for b in range(_NBUF):
        g_copy(ci0 + b, b).wait()
        o_copy(ci0 + b, b).start()
    for b in range(_NBUF):
        o_copy(ci0 + b, b).wait()


def kernel(idx, action_embedding):
    table_pad = jnp.pad(action_embedding, ((0, 0), (0, _DP - 64)))
    junk = jnp.broadcast_to(
        (jnp.arange(_N0, dtype=idx.dtype) % 1000000)[:, None], (_N0, _N1P - _N1)
    )
    idx_pad = jnp.concatenate([idx, junk], axis=1).reshape(-1)
    flat = _gather_kernel(idx_pad, table_pad)
    return flat.reshape(_N0, _N1P, _DP)[:, :_N1, :64]
